# Initial kernel scaffold; baseline (speedup 1.0000x reference)
import jax
import jax.numpy as jnp
from jax.experimental import pallas as pl


def kernel(x, nb, W_pca, b_pca, param_p, W_mlp, b_mlp, Wd, bd, Wc1, bc1, Wc2, bc2, fake_node, fake_cap, real_node, real_cap):
    raise NotImplementedError("write your pallas kernel here")


# speedup vs baseline: 1.9826x; 1.9826x over previous
import functools

import jax
import jax.numpy as jnp
from jax import lax
from jax.experimental import pallas as pl
from jax.experimental.pallas import tpu as pltpu
from jax.experimental.pallas import tpu_sc as plsc

_N = 50000
_M = 16
_D = 64
_K = 8
_DD = 8
_NFEAT = 128
_NCLASS = 16
_NHID2 = 4
_NS = 160
_EPS = 1e-12

_B = 1000
_RB = 1000

_NW = 32
_CHUNK = 200
_NCHUNKS = (_N * _M) // _CHUNK
_CPW = _NCHUNKS // _NW


def _seg_mats(width):
    g = width // _DD
    lane = lax.broadcasted_iota(jnp.int32, (width, g), 0)
    col = lax.broadcasted_iota(jnp.int32, (width, g), 1)
    S = (lane // _DD == col).astype(jnp.float32)
    row = lax.broadcasted_iota(jnp.int32, (g, width), 0)
    lane2 = lax.broadcasted_iota(jnp.int32, (g, width), 1)
    E = (lane2 // _DD == row).astype(jnp.float32)
    return S, E


def _gdot(a, b):
    return jnp.dot(a, b, preferred_element_type=jnp.float32,
                   precision=lax.Precision.HIGHEST)


def _gnorm(u):
    S, E = _seg_mats(u.shape[-1])
    s = _gdot(u * u, S)
    inv = 1.0 / jnp.maximum(jnp.sqrt(s), _EPS)
    return u * _gdot(inv, E)


def _pad128(v):
    return jnp.concatenate([v, jnp.zeros_like(v)], axis=-1)


def _pca_body(x_ref, w_ref, b_ref, h_ref, hn_ref):
    h = jnp.maximum(_gdot(x_ref[...], w_ref[...]) + b_ref[...], 0.0)
    h_ref[...] = h
    hn_ref[...] = _pad128(_gnorm(h))


def _routing_body(param_ref, z_ref, u0_ref, h_ref, hn_ref):
    param = param_ref[0]
    z = z_ref[:, :, : _D]
    ub = u0_ref[:, : _D]
    S, E = _seg_mats(_D)

    c0 = param / 16.0 + (1.0 - param) / 8.0
    u = c0 * jnp.sum(z, axis=1) + ub
    u = _gnorm(u)

    for it in range(1, 3):
        zu = z * u[:, None, :]
        p = _gdot(zu.reshape(_B * _M, _D), S).reshape(_B, _M, _K)
        ep = jnp.exp(p)
        p1 = ep / jnp.sum(ep, axis=1, keepdims=True)
        p2 = ep / jnp.sum(ep, axis=2, keepdims=True)
        w = param * p1 + (1.0 - param) * p2
        we = _gdot(w.reshape(_B * _M, _K), E).reshape(_B, _M, _D)
        u = jnp.sum(z * we, axis=1) + ub
        if it < 2:
            u = _gnorm(u)

    h = jnp.maximum(u, 0.0)
    h_ref[...] = h
    hn_ref[...] = _pad128(_gnorm(h))


def _logit_body(h_ref, w_ref, b_ref, out_ref):
    logit = _gdot(h_ref[...], w_ref[...]) + b_ref[...]
    m = jnp.max(logit, axis=-1, keepdims=True)
    e = logit - m
    lse = jnp.log(jnp.sum(jnp.exp(e), axis=-1, keepdims=True))
    out_ref[...] = e - lse


def _loss_body(fs_ref, rs_ref, ohf_ref, ohr_ref, wd_ref, bd_ref,
               wc1_ref, bc1_ref, wc2_ref, bc2_ref, out_ref):
    hf = jnp.maximum(_gdot(fs_ref[...], wd_ref[...]) + bd_ref[...], 0.0)
    hr = jnp.maximum(_gdot(rs_ref[...], wd_ref[...]) + bd_ref[...], 0.0)
    d_fake = _gdot(hf, wc1_ref[...]) + bc1_ref[...]
    prob = _gdot(hr, wc2_ref[...]) + bc2_ref[...]

    t = -d_fake
    sp = jnp.maximum(t, 0.0) + jnp.log(1.0 + jnp.exp(-jnp.abs(t)))
    lane = lax.broadcasted_iota(jnp.int32, (1, _K), 1)
    g = jnp.sum(sp * (lane == 0).astype(jnp.float32)) / _NS

    m = jnp.max(prob, axis=-1, keepdims=True)
    e = prob - m
    ls = e - jnp.log(jnp.sum(jnp.exp(e), axis=-1, keepdims=True))
    cls_r = -jnp.sum(ls * ohr_ref[...]) / _NS
    cls_f = -jnp.sum(ls * ohf_ref[...]) / _NS

    out_ref[...] = (jnp.where(lane == 0, g + cls_r, 0.0)
                    + jnp.where(lane == 1, g + cls_f, 0.0))


_pca = pl.pallas_call(
    _pca_body,
    grid=(_N // _RB,),
    in_specs=[
        pl.BlockSpec((_RB, _NFEAT), lambda i: (i, 0)),
        pl.BlockSpec((_NFEAT, _D), lambda i: (0, 0)),
        pl.BlockSpec((1, _D), lambda i: (0, 0)),
    ],
    out_specs=[
        pl.BlockSpec((_RB, _D), lambda i: (i, 0)),
        pl.BlockSpec((_RB, 128), lambda i: (i, 0)),
    ],
    out_shape=[
        jax.ShapeDtypeStruct((_N, _D), jnp.float32),
        jax.ShapeDtypeStruct((_N, 128), jnp.float32),
    ],
    compiler_params=pltpu.CompilerParams(dimension_semantics=("parallel",)),
)


_routing = pl.pallas_call(
    _routing_body,
    grid=(_N // _B,),
    in_specs=[
        pl.BlockSpec(memory_space=pltpu.SMEM),
        pl.BlockSpec((_B, _M, 128), lambda i: (i, 0, 0)),
        pl.BlockSpec((_B, 128), lambda i: (i, 0)),
    ],
    out_specs=[
        pl.BlockSpec((_B, _D), lambda i: (i, 0)),
        pl.BlockSpec((_B, 128), lambda i: (i, 0)),
    ],
    out_shape=[
        jax.ShapeDtypeStruct((_N, _D), jnp.float32),
        jax.ShapeDtypeStruct((_N, 128), jnp.float32),
    ],
    compiler_params=pltpu.CompilerParams(dimension_semantics=("parallel",)),
)


_logit = pl.pallas_call(
    _logit_body,
    grid=(_N // _RB,),
    in_specs=[
        pl.BlockSpec((_RB, _D), lambda i: (i, 0)),
        pl.BlockSpec((_D, _NCLASS), lambda i: (0, 0)),
        pl.BlockSpec((1, _NCLASS), lambda i: (0, 0)),
    ],
    out_specs=pl.BlockSpec((_RB, _NCLASS), lambda i: (i, 0)),
    out_shape=jax.ShapeDtypeStruct((_N, _NCLASS), jnp.float32),
    compiler_params=pltpu.CompilerParams(dimension_semantics=("parallel",)),
)


_loss = pl.pallas_call(
    _loss_body,
    out_shape=jax.ShapeDtypeStruct((1, _K), jnp.float32),
)


_sc_gather_cached = None


def _get_sc_gather():
    global _sc_gather_cached
    if _sc_gather_cached is not None:
        return _sc_gather_cached

    @functools.partial(
        pl.kernel,
        mesh=plsc.VectorSubcoreMesh(core_axis_name="c", subcore_axis_name="s"),
        out_type=jax.ShapeDtypeStruct((_N * _M, 128), jnp.float32),
        scratch_types=[
            pltpu.VMEM((_CHUNK,), jnp.int32),
            pltpu.VMEM((_CHUNK, 128), jnp.float32),
            pltpu.SemaphoreType.DMA,
        ],
    )
    def _sc_gather(tab_hbm, idx_hbm, out_hbm, idx_v, rows_v, sem):
        wid = lax.axis_index("s") * 2 + lax.axis_index("c")

        def body(i, carry):
            chunk = wid * _CPW + i
            base = chunk * _CHUNK
            pltpu.sync_copy(idx_hbm.at[pl.ds(base, _CHUNK)], idx_v)
            pltpu.async_copy(tab_hbm.at[idx_v], rows_v, sem).wait()
            pltpu.sync_copy(rows_v, out_hbm.at[pl.ds(base, _CHUNK)])
            return carry

        lax.fori_loop(0, _CPW, body, 0)

    _sc_gather_cached = _sc_gather
    return _sc_gather


def kernel(x, nb, W_pca, b_pca, param_p, W_mlp, b_mlp, Wd, bd, Wc1, bc1,
           Wc2, bc2, fake_node, fake_cap, real_node, real_cap):
    f32 = jnp.float32
    param = jax.nn.sigmoid(param_p.astype(f32))

    h0, hn = _pca(x, W_pca, b_pca.reshape(1, _D))
    idx = nb.astype(jnp.int32).reshape(-1)

    sc_gather = _get_sc_gather()
    h = h0
    for _ in range(2):
        z = sc_gather(hn, idx)
        h, hn = _routing(param, z.reshape(_N, _M, 128), hn)

    out1 = _logit(h, W_mlp, b_mlp.reshape(1, _NCLASS))

    fs = h0.reshape(_N, _K, _DD)[fake_node, fake_cap]
    rs = h.reshape(_N, _K, _DD)[real_node, real_cap]
    ohf = jax.nn.one_hot(fake_cap, _K, dtype=f32)
    ohr = jax.nn.one_hot(real_cap, _K, dtype=f32)
    wc1p = jnp.pad(Wc1, ((0, 0), (0, _K - 1)))
    bc1p = jnp.pad(bc1, (0, _K - 1)).reshape(1, _K)

    lo = _loss(fs, rs, ohf, ohr, Wd, bd.reshape(1, _NHID2), wc1p, bc1p,
               Wc2, bc2.reshape(1, _K))
    return (out1, lo[0, 0], lo[0, 1], h)


# speedup vs baseline: 3.8811x; 1.9576x over previous
import functools

import jax
import jax.numpy as jnp
import numpy as np
from jax import lax
from jax.experimental import pallas as pl
from jax.experimental.pallas import tpu as pltpu
from jax.experimental.pallas import tpu_sc as plsc

_N = 50000
_M = 16
_D = 64
_K = 8
_DD = 8
_NFEAT = 128
_NCLASS = 16
_NHID2 = 4
_NS = 160
_EPS = 1e-12

_B = 400
_RB = 1000
_ZW = _M * 128

_NW = 32
_CHUNK = 200
_NCHUNKS = (_N * _M) // _CHUNK
_CPW = _NCHUNKS // _NW


def _routing_mats():
    l = np.arange(_ZW)
    m = l // 128
    c = l % 128
    valid = c < _D
    j_of_l = m * _K + c // _DD
    SS = np.zeros((_ZW, 128), np.float32)
    SS[l[valid], j_of_l[valid]] = 1.0
    WE = SS.T.copy()
    T = np.zeros((_D, _ZW), np.float32)
    T[c[valid], l[valid]] = 1.0
    R = T.T.copy()
    j = np.arange(128)
    A1 = (j[:, None] % _K == j[None, :] % _K).astype(np.float32)
    A2 = (j[:, None] // _K == j[None, :] // _K).astype(np.float32)
    return SS, WE, T, R, A1, A2


_SS, _WE, _T, _R, _A1, _A2 = _routing_mats()


def _seg_mats(width):
    g = width // _DD
    lane = lax.broadcasted_iota(jnp.int32, (width, g), 0)
    col = lax.broadcasted_iota(jnp.int32, (width, g), 1)
    S = (lane // _DD == col).astype(jnp.float32)
    row = lax.broadcasted_iota(jnp.int32, (g, width), 0)
    lane2 = lax.broadcasted_iota(jnp.int32, (g, width), 1)
    E = (lane2 // _DD == row).astype(jnp.float32)
    return S, E


def _gdot(a, b):
    return jnp.dot(a, b, preferred_element_type=jnp.float32,
                   precision=lax.Precision.HIGHEST)


def _ddot(a, b):
    return jnp.dot(a, b, preferred_element_type=jnp.float32)


def _gnorm(u):
    S, E = _seg_mats(u.shape[-1])
    s = _gdot(u * u, S)
    inv = 1.0 / jnp.maximum(jnp.sqrt(s), _EPS)
    return u * _gdot(inv, E)


def _pad128(v):
    return jnp.concatenate([v, jnp.zeros_like(v)], axis=-1)


def _pca_body(x_ref, w_ref, b_ref, h_ref, hn_ref):
    h = jnp.maximum(_gdot(x_ref[...], w_ref[...]) + b_ref[...], 0.0)
    h_ref[...] = h
    hn_ref[...] = _pad128(_gnorm(h))


def _routing_body(param_ref, z_ref, u0_ref, ss_ref, we_ref, t_ref, r_ref,
                  a1_ref, a2_ref, h_ref, hn_ref):
    param = param_ref[0]
    z = z_ref[...]
    ub = u0_ref[:, : _D]
    SS = ss_ref[...]
    WE = we_ref[...]
    T = t_ref[...]
    R = r_ref[...]
    A1 = a1_ref[...]
    A2 = a2_ref[...]

    c0 = param / 16.0 + (1.0 - param) / 8.0
    u = c0 * _ddot(z, R) + ub
    u = _gnorm(u)

    for it in range(1, 3):
        ut = _ddot(u, T)
        p = _ddot(z * ut, SS)
        ep = jnp.exp(p)
        d1 = _ddot(ep, A1)
        d2 = _ddot(ep, A2)
        w = param * (ep / d1) + (1.0 - param) * (ep / d2)
        we = _ddot(w, WE)
        u = _ddot(z * we, R) + ub
        if it < 2:
            u = _gnorm(u)

    h = jnp.maximum(u, 0.0)
    h_ref[...] = h
    hn_ref[...] = _pad128(_gnorm(h))


def _logit_body(h_ref, w_ref, b_ref, out_ref):
    logit = _gdot(h_ref[...], w_ref[...]) + b_ref[...]
    m = jnp.max(logit, axis=-1, keepdims=True)
    e = logit - m
    lse = jnp.log(jnp.sum(jnp.exp(e), axis=-1, keepdims=True))
    out_ref[...] = e - lse


def _loss_body(fs_ref, rs_ref, ohf_ref, ohr_ref, wd_ref, bd_ref,
               wc1_ref, bc1_ref, wc2_ref, bc2_ref, out_ref):
    hf = jnp.maximum(_gdot(fs_ref[...], wd_ref[...]) + bd_ref[...], 0.0)
    hr = jnp.maximum(_gdot(rs_ref[...], wd_ref[...]) + bd_ref[...], 0.0)
    d_fake = _gdot(hf, wc1_ref[...]) + bc1_ref[...]
    prob = _gdot(hr, wc2_ref[...]) + bc2_ref[...]

    t = -d_fake
    sp = jnp.maximum(t, 0.0) + jnp.log(1.0 + jnp.exp(-jnp.abs(t)))
    lane = lax.broadcasted_iota(jnp.int32, (1, _K), 1)
    g = jnp.sum(sp * (lane == 0).astype(jnp.float32)) / _NS

    m = jnp.max(prob, axis=-1, keepdims=True)
    e = prob - m
    ls = e - jnp.log(jnp.sum(jnp.exp(e), axis=-1, keepdims=True))
    cls_r = -jnp.sum(ls * ohr_ref[...]) / _NS
    cls_f = -jnp.sum(ls * ohf_ref[...]) / _NS

    out_ref[...] = (jnp.where(lane == 0, g + cls_r, 0.0)
                    + jnp.where(lane == 1, g + cls_f, 0.0))


_pca = pl.pallas_call(
    _pca_body,
    grid=(_N // _RB,),
    in_specs=[
        pl.BlockSpec((_RB, _NFEAT), lambda i: (i, 0)),
        pl.BlockSpec((_NFEAT, _D), lambda i: (0, 0)),
        pl.BlockSpec((1, _D), lambda i: (0, 0)),
    ],
    out_specs=[
        pl.BlockSpec((_RB, _D), lambda i: (i, 0)),
        pl.BlockSpec((_RB, 128), lambda i: (i, 0)),
    ],
    out_shape=[
        jax.ShapeDtypeStruct((_N, _D), jnp.float32),
        jax.ShapeDtypeStruct((_N, 128), jnp.float32),
    ],
    compiler_params=pltpu.CompilerParams(dimension_semantics=("parallel",)),
)


_routing = pl.pallas_call(
    _routing_body,
    grid=(_N // _B,),
    in_specs=[
        pl.BlockSpec(memory_space=pltpu.SMEM),
        pl.BlockSpec((_B, _ZW), lambda i: (i, 0)),
        pl.BlockSpec((_B, 128), lambda i: (i, 0)),
        pl.BlockSpec((_ZW, 128), lambda i: (0, 0)),
        pl.BlockSpec((128, _ZW), lambda i: (0, 0)),
        pl.BlockSpec((_D, _ZW), lambda i: (0, 0)),
        pl.BlockSpec((_ZW, _D), lambda i: (0, 0)),
        pl.BlockSpec((128, 128), lambda i: (0, 0)),
        pl.BlockSpec((128, 128), lambda i: (0, 0)),
    ],
    out_specs=[
        pl.BlockSpec((_B, _D), lambda i: (i, 0)),
        pl.BlockSpec((_B, 128), lambda i: (i, 0)),
    ],
    out_shape=[
        jax.ShapeDtypeStruct((_N, _D), jnp.float32),
        jax.ShapeDtypeStruct((_N, 128), jnp.float32),
    ],
    compiler_params=pltpu.CompilerParams(dimension_semantics=("parallel",)),
)


_logit = pl.pallas_call(
    _logit_body,
    grid=(_N // _RB,),
    in_specs=[
        pl.BlockSpec((_RB, _D), lambda i: (i, 0)),
        pl.BlockSpec((_D, _NCLASS), lambda i: (0, 0)),
        pl.BlockSpec((1, _NCLASS), lambda i: (0, 0)),
    ],
    out_specs=pl.BlockSpec((_RB, _NCLASS), lambda i: (i, 0)),
    out_shape=jax.ShapeDtypeStruct((_N, _NCLASS), jnp.float32),
    compiler_params=pltpu.CompilerParams(dimension_semantics=("parallel",)),
)


_loss = pl.pallas_call(
    _loss_body,
    out_shape=jax.ShapeDtypeStruct((1, _K), jnp.float32),
)


_sc_gather_cached = None


def _get_sc_gather():
    global _sc_gather_cached
    if _sc_gather_cached is not None:
        return _sc_gather_cached

    @functools.partial(
        pl.kernel,
        mesh=plsc.VectorSubcoreMesh(core_axis_name="c", subcore_axis_name="s"),
        out_type=jax.ShapeDtypeStruct((_N * _M, 128), jnp.float32),
        scratch_types=[
            pltpu.VMEM((_CHUNK,), jnp.int32),
            pltpu.VMEM((_CHUNK, 128), jnp.float32),
            pltpu.SemaphoreType.DMA,
        ],
    )
    def _sc_gather(tab_hbm, idx_hbm, out_hbm, idx_v, rows_v, sem):
        wid = lax.axis_index("s") * 2 + lax.axis_index("c")

        def body(i, carry):
            chunk = wid * _CPW + i
            base = chunk * _CHUNK
            pltpu.sync_copy(idx_hbm.at[pl.ds(base, _CHUNK)], idx_v)
            pltpu.async_copy(tab_hbm.at[idx_v], rows_v, sem).wait()
            pltpu.sync_copy(rows_v, out_hbm.at[pl.ds(base, _CHUNK)])
            return carry

        lax.fori_loop(0, _CPW, body, 0)

    _sc_gather_cached = _sc_gather
    return _sc_gather


def kernel(x, nb, W_pca, b_pca, param_p, W_mlp, b_mlp, Wd, bd, Wc1, bc1,
           Wc2, bc2, fake_node, fake_cap, real_node, real_cap):
    f32 = jnp.float32
    param = jax.nn.sigmoid(param_p.astype(f32))

    h0, hn = _pca(x, W_pca, b_pca.reshape(1, _D))
    idx = nb.astype(jnp.int32).reshape(-1)

    sc_gather = _get_sc_gather()
    h = h0
    for _ in range(2):
        z = sc_gather(hn, idx)
        h, hn = _routing(param, z.reshape(_N, _ZW), hn,
                         _SS, _WE, _T, _R, _A1, _A2)

    out1 = _logit(h, W_mlp, b_mlp.reshape(1, _NCLASS))

    fs = h0.reshape(_N, _K, _DD)[fake_node, fake_cap]
    rs = h.reshape(_N, _K, _DD)[real_node, real_cap]
    ohf = jax.nn.one_hot(fake_cap, _K, dtype=f32)
    ohr = jax.nn.one_hot(real_cap, _K, dtype=f32)
    wc1p = jnp.pad(Wc1, ((0, 0), (0, _K - 1)))
    bc1p = jnp.pad(bc1, (0, _K - 1)).reshape(1, _K)

    lo = _loss(fs, rs, ohf, ohr, Wd, bd.reshape(1, _NHID2), wc1p, bc1p,
               Wc2, bc2.reshape(1, _K))
    return (out1, lo[0, 0], lo[0, 1], h)


# speedup vs baseline: 6.3480x; 1.6356x over previous
import functools

import jax
import jax.numpy as jnp
import numpy as np
from jax import lax
from jax.experimental import pallas as pl
from jax.experimental.pallas import tpu as pltpu
from jax.experimental.pallas import tpu_sc as plsc

_N = 50000
_M = 16
_D = 64
_K = 8
_DD = 8
_NFEAT = 128
_NCLASS = 16
_NHID2 = 4
_NS = 160
_EPS = 1e-12

_B = 400
_RB = 1000
_ZW = _M * _D

_NW = 32
_CHUNK = 1000
_NCHUNKS = (_N * _M) // _CHUNK
_CPW = _NCHUNKS // _NW


def _routing_mats():
    l = np.arange(_ZW)
    m = l // _D
    c = l % _D
    j_of_l = m * _K + c // _DD
    SS = np.zeros((_ZW, 128), np.float32)
    SS[l, j_of_l] = 1.0
    WE = SS.T.copy()
    T = np.zeros((_D, _ZW), np.float32)
    T[c, l] = 1.0
    R = T.T.copy()
    j = np.arange(128)
    A1 = (j[:, None] % _K == j[None, :] % _K).astype(np.float32)
    A2 = (j[:, None] // _K == j[None, :] // _K).astype(np.float32)
    return SS, WE, T, R, A1, A2


_SS, _WE, _T, _R, _A1, _A2 = _routing_mats()


def _seg_mats(width):
    g = width // _DD
    lane = lax.broadcasted_iota(jnp.int32, (width, g), 0)
    col = lax.broadcasted_iota(jnp.int32, (width, g), 1)
    S = (lane // _DD == col).astype(jnp.float32)
    row = lax.broadcasted_iota(jnp.int32, (g, width), 0)
    lane2 = lax.broadcasted_iota(jnp.int32, (g, width), 1)
    E = (lane2 // _DD == row).astype(jnp.float32)
    return S, E


def _gdot(a, b):
    return jnp.dot(a, b, preferred_element_type=jnp.float32,
                   precision=lax.Precision.HIGHEST)


def _ddot(a, b):
    return jnp.dot(a, b, preferred_element_type=jnp.float32)


def _gnorm(u):
    S, E = _seg_mats(u.shape[-1])
    s = _gdot(u * u, S)
    inv = 1.0 / jnp.maximum(jnp.sqrt(s), _EPS)
    return u * _gdot(inv, E)


def _pad128(v):
    return jnp.concatenate([v, jnp.zeros_like(v)], axis=-1)


def _pca_body(x_ref, w_ref, b_ref, h_ref, hn_ref):
    h = jnp.maximum(_gdot(x_ref[...], w_ref[...]) + b_ref[...], 0.0)
    h_ref[...] = h
    hn_ref[...] = _gnorm(h)


def _routing_body(param_ref, z_ref, u0_ref, ss_ref, we_ref, t_ref, r_ref,
                  a1_ref, a2_ref, h_ref, hn_ref):
    param = param_ref[0]
    z = z_ref[...]
    ub = u0_ref[...]
    SS = ss_ref[...]
    WE = we_ref[...]
    T = t_ref[...]
    R = r_ref[...]
    A1 = a1_ref[...]
    A2 = a2_ref[...]

    c0 = param / 16.0 + (1.0 - param) / 8.0
    u = c0 * _ddot(z, R) + ub
    u = _gnorm(u)

    for it in range(1, 3):
        ut = _ddot(u, T)
        p = _ddot(z * ut, SS)
        ep = jnp.exp(p)
        d1 = _ddot(ep, A1)
        d2 = _ddot(ep, A2)
        w = param * (ep / d1) + (1.0 - param) * (ep / d2)
        we = _ddot(w, WE)
        u = _ddot(z * we, R) + ub
        if it < 2:
            u = _gnorm(u)

    h = jnp.maximum(u, 0.0)
    h_ref[...] = h
    hn_ref[...] = _gnorm(h)


def _logit_body(h_ref, w_ref, b_ref, out_ref):
    logit = _gdot(h_ref[...], w_ref[...]) + b_ref[...]
    m = jnp.max(logit, axis=-1, keepdims=True)
    e = logit - m
    lse = jnp.log(jnp.sum(jnp.exp(e), axis=-1, keepdims=True))
    out_ref[...] = e - lse


def _loss_body(fs_ref, rs_ref, ohf_ref, ohr_ref, wd_ref, bd_ref,
               wc1_ref, bc1_ref, wc2_ref, bc2_ref, out_ref):
    hf = jnp.maximum(_gdot(fs_ref[...], wd_ref[...]) + bd_ref[...], 0.0)
    hr = jnp.maximum(_gdot(rs_ref[...], wd_ref[...]) + bd_ref[...], 0.0)
    d_fake = _gdot(hf, wc1_ref[...]) + bc1_ref[...]
    prob = _gdot(hr, wc2_ref[...]) + bc2_ref[...]

    t = -d_fake
    sp = jnp.maximum(t, 0.0) + jnp.log(1.0 + jnp.exp(-jnp.abs(t)))
    lane = lax.broadcasted_iota(jnp.int32, (1, _K), 1)
    g = jnp.sum(sp * (lane == 0).astype(jnp.float32)) / _NS

    m = jnp.max(prob, axis=-1, keepdims=True)
    e = prob - m
    ls = e - jnp.log(jnp.sum(jnp.exp(e), axis=-1, keepdims=True))
    cls_r = -jnp.sum(ls * ohr_ref[...]) / _NS
    cls_f = -jnp.sum(ls * ohf_ref[...]) / _NS

    out_ref[...] = (jnp.where(lane == 0, g + cls_r, 0.0)
                    + jnp.where(lane == 1, g + cls_f, 0.0))


_pca = pl.pallas_call(
    _pca_body,
    grid=(_N // _RB,),
    in_specs=[
        pl.BlockSpec((_RB, _NFEAT), lambda i: (i, 0)),
        pl.BlockSpec((_NFEAT, _D), lambda i: (0, 0)),
        pl.BlockSpec((1, _D), lambda i: (0, 0)),
    ],
    out_specs=[
        pl.BlockSpec((_RB, _D), lambda i: (i, 0)),
        pl.BlockSpec((_RB, _D), lambda i: (i, 0)),
    ],
    out_shape=[
        jax.ShapeDtypeStruct((_N, _D), jnp.float32),
        jax.ShapeDtypeStruct((_N, _D), jnp.float32),
    ],
    compiler_params=pltpu.CompilerParams(dimension_semantics=("parallel",)),
)


_routing = pl.pallas_call(
    _routing_body,
    grid=(_N // _B,),
    in_specs=[
        pl.BlockSpec(memory_space=pltpu.SMEM),
        pl.BlockSpec((_B, _ZW), lambda i: (i, 0)),
        pl.BlockSpec((_B, _D), lambda i: (i, 0)),
        pl.BlockSpec((_ZW, 128), lambda i: (0, 0)),
        pl.BlockSpec((128, _ZW), lambda i: (0, 0)),
        pl.BlockSpec((_D, _ZW), lambda i: (0, 0)),
        pl.BlockSpec((_ZW, _D), lambda i: (0, 0)),
        pl.BlockSpec((128, 128), lambda i: (0, 0)),
        pl.BlockSpec((128, 128), lambda i: (0, 0)),
    ],
    out_specs=[
        pl.BlockSpec((_B, _D), lambda i: (i, 0)),
        pl.BlockSpec((_B, _D), lambda i: (i, 0)),
    ],
    out_shape=[
        jax.ShapeDtypeStruct((_N, _D), jnp.float32),
        jax.ShapeDtypeStruct((_N, _D), jnp.float32),
    ],
    compiler_params=pltpu.CompilerParams(dimension_semantics=("parallel",)),
)


_logit = pl.pallas_call(
    _logit_body,
    grid=(_N // _RB,),
    in_specs=[
        pl.BlockSpec((_RB, _D), lambda i: (i, 0)),
        pl.BlockSpec((_D, _NCLASS), lambda i: (0, 0)),
        pl.BlockSpec((1, _NCLASS), lambda i: (0, 0)),
    ],
    out_specs=pl.BlockSpec((_RB, _NCLASS), lambda i: (i, 0)),
    out_shape=jax.ShapeDtypeStruct((_N, _NCLASS), jnp.float32),
    compiler_params=pltpu.CompilerParams(dimension_semantics=("parallel",)),
)


_loss = pl.pallas_call(
    _loss_body,
    out_shape=jax.ShapeDtypeStruct((1, _K), jnp.float32),
)


_sc_gather_cached = None


def _get_sc_gather():
    global _sc_gather_cached
    if _sc_gather_cached is not None:
        return _sc_gather_cached

    @functools.partial(
        pl.kernel,
        mesh=plsc.VectorSubcoreMesh(core_axis_name="c", subcore_axis_name="s"),
        out_type=jax.ShapeDtypeStruct((_N * _M, _D), jnp.float32),
        scratch_types=[
            pltpu.VMEM((_CHUNK,), jnp.int32),
            pltpu.VMEM((_CHUNK, _D), jnp.float32),
            pltpu.SemaphoreType.DMA,
        ],
        compiler_params=pltpu.CompilerParams(use_tc_tiling_on_sc=False),
    )
    def _sc_gather(tab_hbm, idx_hbm, out_hbm, idx_v, rows_v, sem):
        wid = lax.axis_index("s") * 2 + lax.axis_index("c")

        def body(i, carry):
            chunk = wid * _CPW + i
            base = chunk * _CHUNK
            pltpu.sync_copy(idx_hbm.at[pl.ds(base, _CHUNK)], idx_v)
            pltpu.async_copy(tab_hbm.at[idx_v], rows_v, sem).wait()
            pltpu.sync_copy(rows_v, out_hbm.at[pl.ds(base, _CHUNK)])
            return carry

        lax.fori_loop(0, _CPW, body, 0)

    _sc_gather_cached = _sc_gather
    return _sc_gather


def kernel(x, nb, W_pca, b_pca, param_p, W_mlp, b_mlp, Wd, bd, Wc1, bc1,
           Wc2, bc2, fake_node, fake_cap, real_node, real_cap):
    f32 = jnp.float32
    param = jax.nn.sigmoid(param_p.astype(f32))

    h0, hn = _pca(x, W_pca, b_pca.reshape(1, _D))
    idx = nb.astype(jnp.int32).reshape(-1)

    sc_gather = _get_sc_gather()
    h = h0
    for _ in range(2):
        z = sc_gather(hn, idx)
        h, hn = _routing(param, z.reshape(_N, _ZW), hn,
                         _SS, _WE, _T, _R, _A1, _A2)

    out1 = _logit(h, W_mlp, b_mlp.reshape(1, _NCLASS))

    fs = h0.reshape(_N, _K, _DD)[fake_node, fake_cap]
    rs = h.reshape(_N, _K, _DD)[real_node, real_cap]
    ohf = jax.nn.one_hot(fake_cap, _K, dtype=f32)
    ohr = jax.nn.one_hot(real_cap, _K, dtype=f32)
    wc1p = jnp.pad(Wc1, ((0, 0), (0, _K - 1)))
    bc1p = jnp.pad(bc1, (0, _K - 1)).reshape(1, _K)

    lo = _loss(fs, rs, ohf, ohr, Wd, bd.reshape(1, _NHID2), wc1p, bc1p,
               Wc2, bc2.reshape(1, _K))
    return (out1, lo[0, 0], lo[0, 1], h)


# speedup vs baseline: 6.4116x; 1.0100x over previous
import functools

import jax
import jax.numpy as jnp
import numpy as np
from jax import lax
from jax.experimental import pallas as pl
from jax.experimental.pallas import tpu as pltpu
from jax.experimental.pallas import tpu_sc as plsc

_N = 50000
_M = 16
_D = 64
_K = 8
_DD = 8
_NFEAT = 128
_NCLASS = 16
_NHID2 = 4
_NS = 160
_EPS = 1e-12

_B = 1000
_RB = 1000
_ZW = _M * _D

_NW = 32
_CHUNK = 1000
_NCHUNKS = (_N * _M) // _CHUNK
_CPW = _NCHUNKS // _NW


def _routing_mats():
    l = np.arange(_ZW)
    m = l // _D
    c = l % _D
    j_of_l = m * _K + c // _DD
    SS = np.zeros((_ZW, 128), np.float32)
    SS[l, j_of_l] = 1.0
    WE = SS.T.copy()
    T = np.zeros((_D, _ZW), np.float32)
    T[c, l] = 1.0
    R = T.T.copy()
    j = np.arange(128)
    A1 = (j[:, None] % _K == j[None, :] % _K).astype(np.float32)
    A2 = (j[:, None] // _K == j[None, :] // _K).astype(np.float32)
    return SS, WE, T, R, A1, A2


_SS, _WE, _T, _R, _A1, _A2 = _routing_mats()


def _seg_mats(width):
    g = width // _DD
    lane = lax.broadcasted_iota(jnp.int32, (width, g), 0)
    col = lax.broadcasted_iota(jnp.int32, (width, g), 1)
    S = (lane // _DD == col).astype(jnp.float32)
    row = lax.broadcasted_iota(jnp.int32, (g, width), 0)
    lane2 = lax.broadcasted_iota(jnp.int32, (g, width), 1)
    E = (lane2 // _DD == row).astype(jnp.float32)
    return S, E


def _gdot(a, b):
    return jnp.dot(a, b, preferred_element_type=jnp.float32,
                   precision=lax.Precision.HIGHEST)


def _ddot(a, b):
    return jnp.dot(a, b, preferred_element_type=jnp.float32)


def _gnorm(u):
    S, E = _seg_mats(u.shape[-1])
    s = _gdot(u * u, S)
    inv = 1.0 / jnp.maximum(jnp.sqrt(s), _EPS)
    return u * _gdot(inv, E)


def _pad128(v):
    return jnp.concatenate([v, jnp.zeros_like(v)], axis=-1)


def _pca_body(x_ref, w_ref, b_ref, h_ref, hn_ref):
    h = jnp.maximum(_gdot(x_ref[...], w_ref[...]) + b_ref[...], 0.0)
    h_ref[...] = h
    hn_ref[...] = _gnorm(h)


def _routing_body(param_ref, z_ref, u0_ref, ss_ref, we_ref, t_ref, r_ref,
                  a1_ref, a2_ref, h_ref, hn_ref):
    param = param_ref[0]
    z = z_ref[...]
    ub = u0_ref[...]
    SS = ss_ref[...]
    WE = we_ref[...]
    T = t_ref[...]
    R = r_ref[...]
    A1 = a1_ref[...]
    A2 = a2_ref[...]

    c0 = param / 16.0 + (1.0 - param) / 8.0
    u = c0 * _ddot(z, R) + ub
    u = _gnorm(u)

    for it in range(1, 3):
        ut = _ddot(u, T)
        p = _ddot(z * ut, SS)
        ep = jnp.exp(p)
        d1 = _ddot(ep, A1)
        d2 = _ddot(ep, A2)
        w = param * (ep / d1) + (1.0 - param) * (ep / d2)
        we = _ddot(w, WE)
        u = _ddot(z * we, R) + ub
        if it < 2:
            u = _gnorm(u)

    h = jnp.maximum(u, 0.0)
    h_ref[...] = h
    hn_ref[...] = _gnorm(h)


def _logit_body(h_ref, w_ref, b_ref, out_ref):
    logit = _gdot(h_ref[...], w_ref[...]) + b_ref[...]
    m = jnp.max(logit, axis=-1, keepdims=True)
    e = logit - m
    lse = jnp.log(jnp.sum(jnp.exp(e), axis=-1, keepdims=True))
    out_ref[...] = e - lse


def _loss_body(fs_ref, rs_ref, ohf_ref, ohr_ref, wd_ref, bd_ref,
               wc1_ref, bc1_ref, wc2_ref, bc2_ref, out_ref):
    hf = jnp.maximum(_gdot(fs_ref[...], wd_ref[...]) + bd_ref[...], 0.0)
    hr = jnp.maximum(_gdot(rs_ref[...], wd_ref[...]) + bd_ref[...], 0.0)
    d_fake = _gdot(hf, wc1_ref[...]) + bc1_ref[...]
    prob = _gdot(hr, wc2_ref[...]) + bc2_ref[...]

    t = -d_fake
    sp = jnp.maximum(t, 0.0) + jnp.log(1.0 + jnp.exp(-jnp.abs(t)))
    lane = lax.broadcasted_iota(jnp.int32, (1, _K), 1)
    g = jnp.sum(sp * (lane == 0).astype(jnp.float32)) / _NS

    m = jnp.max(prob, axis=-1, keepdims=True)
    e = prob - m
    ls = e - jnp.log(jnp.sum(jnp.exp(e), axis=-1, keepdims=True))
    cls_r = -jnp.sum(ls * ohr_ref[...]) / _NS
    cls_f = -jnp.sum(ls * ohf_ref[...]) / _NS

    out_ref[...] = (jnp.where(lane == 0, g + cls_r, 0.0)
                    + jnp.where(lane == 1, g + cls_f, 0.0))


_pca = pl.pallas_call(
    _pca_body,
    grid=(_N // _RB,),
    in_specs=[
        pl.BlockSpec((_RB, _NFEAT), lambda i: (i, 0)),
        pl.BlockSpec((_NFEAT, _D), lambda i: (0, 0)),
        pl.BlockSpec((1, _D), lambda i: (0, 0)),
    ],
    out_specs=[
        pl.BlockSpec((_RB, _D), lambda i: (i, 0)),
        pl.BlockSpec((_RB, _D), lambda i: (i, 0)),
    ],
    out_shape=[
        jax.ShapeDtypeStruct((_N, _D), jnp.float32),
        jax.ShapeDtypeStruct((_N, _D), jnp.float32),
    ],
    compiler_params=pltpu.CompilerParams(dimension_semantics=("parallel",)),
)


_routing = pl.pallas_call(
    _routing_body,
    grid=(_N // _B,),
    in_specs=[
        pl.BlockSpec(memory_space=pltpu.SMEM),
        pl.BlockSpec((_B, _ZW), lambda i: (i, 0)),
        pl.BlockSpec((_B, _D), lambda i: (i, 0)),
        pl.BlockSpec((_ZW, 128), lambda i: (0, 0)),
        pl.BlockSpec((128, _ZW), lambda i: (0, 0)),
        pl.BlockSpec((_D, _ZW), lambda i: (0, 0)),
        pl.BlockSpec((_ZW, _D), lambda i: (0, 0)),
        pl.BlockSpec((128, 128), lambda i: (0, 0)),
        pl.BlockSpec((128, 128), lambda i: (0, 0)),
    ],
    out_specs=[
        pl.BlockSpec((_B, _D), lambda i: (i, 0)),
        pl.BlockSpec((_B, _D), lambda i: (i, 0)),
    ],
    out_shape=[
        jax.ShapeDtypeStruct((_N, _D), jnp.float32),
        jax.ShapeDtypeStruct((_N, _D), jnp.float32),
    ],
    compiler_params=pltpu.CompilerParams(dimension_semantics=("parallel",)),
)


_logit = pl.pallas_call(
    _logit_body,
    grid=(_N // _RB,),
    in_specs=[
        pl.BlockSpec((_RB, _D), lambda i: (i, 0)),
        pl.BlockSpec((_D, _NCLASS), lambda i: (0, 0)),
        pl.BlockSpec((1, _NCLASS), lambda i: (0, 0)),
    ],
    out_specs=pl.BlockSpec((_RB, _NCLASS), lambda i: (i, 0)),
    out_shape=jax.ShapeDtypeStruct((_N, _NCLASS), jnp.float32),
    compiler_params=pltpu.CompilerParams(dimension_semantics=("parallel",)),
)


_loss = pl.pallas_call(
    _loss_body,
    out_shape=jax.ShapeDtypeStruct((1, _K), jnp.float32),
)


_sc_gather_cached = None


def _get_sc_gather():
    global _sc_gather_cached
    if _sc_gather_cached is not None:
        return _sc_gather_cached

    @functools.partial(
        pl.kernel,
        mesh=plsc.VectorSubcoreMesh(core_axis_name="c", subcore_axis_name="s"),
        out_type=jax.ShapeDtypeStruct((_N * _M, _D), jnp.float32),
        scratch_types=[
            pltpu.VMEM((_CHUNK,), jnp.int32),
            pltpu.VMEM((_CHUNK, _D), jnp.float32),
            pltpu.SemaphoreType.DMA,
        ],
        compiler_params=pltpu.CompilerParams(use_tc_tiling_on_sc=False),
    )
    def _sc_gather(tab_hbm, idx_hbm, out_hbm, idx_v, rows_v, sem):
        wid = lax.axis_index("s") * 2 + lax.axis_index("c")

        def body(i, carry):
            chunk = wid * _CPW + i
            base = chunk * _CHUNK
            pltpu.sync_copy(idx_hbm.at[pl.ds(base, _CHUNK)], idx_v)
            pltpu.async_copy(tab_hbm.at[idx_v], rows_v, sem).wait()
            pltpu.sync_copy(rows_v, out_hbm.at[pl.ds(base, _CHUNK)])
            return carry

        lax.fori_loop(0, _CPW, body, 0)

    _sc_gather_cached = _sc_gather
    return _sc_gather


def kernel(x, nb, W_pca, b_pca, param_p, W_mlp, b_mlp, Wd, bd, Wc1, bc1,
           Wc2, bc2, fake_node, fake_cap, real_node, real_cap):
    f32 = jnp.float32
    param = jax.nn.sigmoid(param_p.astype(f32))

    h0, hn = _pca(x, W_pca, b_pca.reshape(1, _D))
    idx = nb.astype(jnp.int32).reshape(-1)

    sc_gather = _get_sc_gather()
    h = h0
    for _ in range(2):
        z = sc_gather(hn, idx)
        h, hn = _routing(param, z.reshape(_N, _ZW), hn,
                         _SS, _WE, _T, _R, _A1, _A2)

    out1 = _logit(h, W_mlp, b_mlp.reshape(1, _NCLASS))

    fs = h0.reshape(_N, _K, _DD)[fake_node, fake_cap]
    rs = h.reshape(_N, _K, _DD)[real_node, real_cap]
    ohf = jax.nn.one_hot(fake_cap, _K, dtype=f32)
    ohr = jax.nn.one_hot(real_cap, _K, dtype=f32)
    wc1p = jnp.pad(Wc1, ((0, 0), (0, _K - 1)))
    bc1p = jnp.pad(bc1, (0, _K - 1)).reshape(1, _K)

    lo = _loss(fs, rs, ohf, ohr, Wd, bd.reshape(1, _NHID2), wc1p, bc1p,
               Wc2, bc2.reshape(1, _K))
    return (out1, lo[0, 0], lo[0, 1], h)


# speedup vs baseline: 9.2498x; 1.4427x over previous
import functools

import jax
import jax.numpy as jnp
import numpy as np
from jax import lax
from jax.experimental import pallas as pl
from jax.experimental.pallas import tpu as pltpu
from jax.experimental.pallas import tpu_sc as plsc

_N = 50000
_M = 16
_D = 64
_K = 8
_DD = 8
_NFEAT = 128
_NCLASS = 16
_NHID2 = 4
_NS = 160
_EPS = 1e-12

_B = 1000
_RB = 1000
_ZW = _M * _D

_NW = 32
_CHUNK = 1000
_NCHUNKS = (_N * _M) // _CHUNK
_CPW = _NCHUNKS // _NW


def _routing_mats():
    l = np.arange(_ZW)
    m = l // _D
    c = l % _D
    j_of_l = m * _K + c // _DD
    SS = np.zeros((_ZW, 128), np.float32)
    SS[l, j_of_l] = 1.0
    WE = SS.T.copy()
    T = np.zeros((_D, _ZW), np.float32)
    T[c, l] = 1.0
    R = T.T.copy()
    j = np.arange(128)
    A1 = (j[:, None] % _K == j[None, :] % _K).astype(np.float32)
    A2 = (j[:, None] // _K == j[None, :] // _K).astype(np.float32)
    return SS, WE, T, R, A1, A2


_SS, _WE, _T, _R, _A1, _A2 = _routing_mats()


def _seg_mats(width):
    g = width // _DD
    lane = lax.broadcasted_iota(jnp.int32, (width, g), 0)
    col = lax.broadcasted_iota(jnp.int32, (width, g), 1)
    S = (lane // _DD == col).astype(jnp.float32)
    row = lax.broadcasted_iota(jnp.int32, (g, width), 0)
    lane2 = lax.broadcasted_iota(jnp.int32, (g, width), 1)
    E = (lane2 // _DD == row).astype(jnp.float32)
    return S, E


def _gdot(a, b):
    return jnp.dot(a, b, preferred_element_type=jnp.float32,
                   precision=lax.Precision.HIGHEST)


def _ddot(a, b):
    return jnp.dot(a, b, preferred_element_type=jnp.float32)


def _gnorm(u):
    S, E = _seg_mats(u.shape[-1])
    s = _ddot(u * u, S)
    inv = 1.0 / jnp.maximum(jnp.sqrt(s), _EPS)
    return u * _ddot(inv, E)


def _pad128(v):
    return jnp.concatenate([v, jnp.zeros_like(v)], axis=-1)


def _pca_body(x_ref, w_ref, b_ref, h_ref, hn_ref):
    h = jnp.maximum(_gdot(x_ref[...], w_ref[...]) + b_ref[...], 0.0)
    h_ref[...] = h
    hn_ref[...] = _gnorm(h)


def _routing_body(param_ref, z_ref, u0_ref, ss_ref, we_ref, t_ref, r_ref,
                  a1_ref, a2_ref, h_ref, hn_ref):
    param = param_ref[0]
    z = z_ref[...]
    ub = u0_ref[...]
    SS = ss_ref[...]
    WE = we_ref[...]
    T = t_ref[...]
    R = r_ref[...]
    A1 = a1_ref[...]
    A2 = a2_ref[...]

    c0 = param / 16.0 + (1.0 - param) / 8.0
    u = c0 * _ddot(z, R) + ub
    u = _gnorm(u)

    for it in range(1, 3):
        ut = _ddot(u, T)
        p = _ddot(z * ut, SS)
        ep = jnp.exp(p)
        d1 = _ddot(ep, A1)
        d2 = _ddot(ep, A2)
        w = param * (ep / d1) + (1.0 - param) * (ep / d2)
        we = _ddot(w, WE)
        u = _ddot(z * we, R) + ub
        if it < 2:
            u = _gnorm(u)

    h = jnp.maximum(u, 0.0)
    h_ref[...] = h
    hn_ref[...] = _gnorm(h)


def _logit_body(h_ref, w_ref, b_ref, out_ref):
    logit = _gdot(h_ref[...], w_ref[...]) + b_ref[...]
    m = jnp.max(logit, axis=-1, keepdims=True)
    e = logit - m
    lse = jnp.log(jnp.sum(jnp.exp(e), axis=-1, keepdims=True))
    out_ref[...] = e - lse


def _loss_body(fs_ref, rs_ref, ohf_ref, ohr_ref, wd_ref, bd_ref,
               wc1_ref, bc1_ref, wc2_ref, bc2_ref, out_ref):
    hf = jnp.maximum(_gdot(fs_ref[...], wd_ref[...]) + bd_ref[...], 0.0)
    hr = jnp.maximum(_gdot(rs_ref[...], wd_ref[...]) + bd_ref[...], 0.0)
    d_fake = _gdot(hf, wc1_ref[...]) + bc1_ref[...]
    prob = _gdot(hr, wc2_ref[...]) + bc2_ref[...]

    t = -d_fake
    sp = jnp.maximum(t, 0.0) + jnp.log(1.0 + jnp.exp(-jnp.abs(t)))
    lane = lax.broadcasted_iota(jnp.int32, (1, _K), 1)
    g = jnp.sum(sp * (lane == 0).astype(jnp.float32)) / _NS

    m = jnp.max(prob, axis=-1, keepdims=True)
    e = prob - m
    ls = e - jnp.log(jnp.sum(jnp.exp(e), axis=-1, keepdims=True))
    cls_r = -jnp.sum(ls * ohr_ref[...]) / _NS
    cls_f = -jnp.sum(ls * ohf_ref[...]) / _NS

    out_ref[...] = (jnp.where(lane == 0, g + cls_r, 0.0)
                    + jnp.where(lane == 1, g + cls_f, 0.0))


_pca = pl.pallas_call(
    _pca_body,
    grid=(_N // _RB,),
    in_specs=[
        pl.BlockSpec((_RB, _NFEAT), lambda i: (i, 0)),
        pl.BlockSpec((_NFEAT, _D), lambda i: (0, 0)),
        pl.BlockSpec((1, _D), lambda i: (0, 0)),
    ],
    out_specs=[
        pl.BlockSpec((_RB, _D), lambda i: (i, 0)),
        pl.BlockSpec((_RB, _D), lambda i: (i, 0)),
    ],
    out_shape=[
        jax.ShapeDtypeStruct((_N, _D), jnp.float32),
        jax.ShapeDtypeStruct((_N, _D), jnp.float32),
    ],
    compiler_params=pltpu.CompilerParams(dimension_semantics=("parallel",)),
)


_routing = pl.pallas_call(
    _routing_body,
    grid=(_N // _B,),
    in_specs=[
        pl.BlockSpec(memory_space=pltpu.SMEM),
        pl.BlockSpec((_B, _ZW), lambda i: (i, 0)),
        pl.BlockSpec((_B, _D), lambda i: (i, 0)),
        pl.BlockSpec((_ZW, 128), lambda i: (0, 0)),
        pl.BlockSpec((128, _ZW), lambda i: (0, 0)),
        pl.BlockSpec((_D, _ZW), lambda i: (0, 0)),
        pl.BlockSpec((_ZW, _D), lambda i: (0, 0)),
        pl.BlockSpec((128, 128), lambda i: (0, 0)),
        pl.BlockSpec((128, 128), lambda i: (0, 0)),
    ],
    out_specs=[
        pl.BlockSpec((_B, _D), lambda i: (i, 0)),
        pl.BlockSpec((_B, _D), lambda i: (i, 0)),
    ],
    out_shape=[
        jax.ShapeDtypeStruct((_N, _D), jnp.float32),
        jax.ShapeDtypeStruct((_N, _D), jnp.float32),
    ],
    compiler_params=pltpu.CompilerParams(dimension_semantics=("parallel",)),
)


_logit = pl.pallas_call(
    _logit_body,
    grid=(_N // _RB,),
    in_specs=[
        pl.BlockSpec((_RB, _D), lambda i: (i, 0)),
        pl.BlockSpec((_D, _NCLASS), lambda i: (0, 0)),
        pl.BlockSpec((1, _NCLASS), lambda i: (0, 0)),
    ],
    out_specs=pl.BlockSpec((_RB, _NCLASS), lambda i: (i, 0)),
    out_shape=jax.ShapeDtypeStruct((_N, _NCLASS), jnp.float32),
    compiler_params=pltpu.CompilerParams(dimension_semantics=("parallel",)),
)


_loss = pl.pallas_call(
    _loss_body,
    out_shape=jax.ShapeDtypeStruct((1, _K), jnp.float32),
)


_sc_gather_cached = None


def _get_sc_gather():
    global _sc_gather_cached
    if _sc_gather_cached is not None:
        return _sc_gather_cached

    @functools.partial(
        pl.kernel,
        mesh=plsc.VectorSubcoreMesh(core_axis_name="c", subcore_axis_name="s"),
        out_type=jax.ShapeDtypeStruct((_N * _M, _D), jnp.float32),
        scratch_types=[
            pltpu.VMEM((_CHUNK,), jnp.int32),
            pltpu.VMEM((_CHUNK, _D), jnp.float32),
            pltpu.SemaphoreType.DMA,
        ],
        compiler_params=pltpu.CompilerParams(use_tc_tiling_on_sc=False),
    )
    def _sc_gather(tab_hbm, idx_hbm, out_hbm, idx_v, rows_v, sem):
        wid = lax.axis_index("s") * 2 + lax.axis_index("c")

        def body(i, carry):
            chunk = wid * _CPW + i
            base = chunk * _CHUNK
            pltpu.sync_copy(idx_hbm.at[pl.ds(base, _CHUNK)], idx_v)
            pltpu.async_copy(tab_hbm.at[idx_v], rows_v, sem).wait()
            pltpu.sync_copy(rows_v, out_hbm.at[pl.ds(base, _CHUNK)])
            return carry

        lax.fori_loop(0, _CPW, body, 0)

    _sc_gather_cached = _sc_gather
    return _sc_gather


def kernel(x, nb, W_pca, b_pca, param_p, W_mlp, b_mlp, Wd, bd, Wc1, bc1,
           Wc2, bc2, fake_node, fake_cap, real_node, real_cap):
    f32 = jnp.float32
    param = jax.nn.sigmoid(param_p.astype(f32))

    h0, hn = _pca(x, W_pca, b_pca.reshape(1, _D))
    idx = nb.astype(jnp.int32).reshape(-1)

    sc_gather = _get_sc_gather()
    h = h0
    for _ in range(2):
        z = sc_gather(hn, idx)
        h, hn = _routing(param, z.reshape(_N, _ZW), hn,
                         _SS, _WE, _T, _R, _A1, _A2)

    out1 = _logit(h, W_mlp, b_mlp.reshape(1, _NCLASS))

    fs = h0.reshape(_N, _K, _DD)[fake_node, fake_cap]
    rs = h.reshape(_N, _K, _DD)[real_node, real_cap]
    ohf = jax.nn.one_hot(fake_cap, _K, dtype=f32)
    ohr = jax.nn.one_hot(real_cap, _K, dtype=f32)
    wc1p = jnp.pad(Wc1, ((0, 0), (0, _K - 1)))
    bc1p = jnp.pad(bc1, (0, _K - 1)).reshape(1, _K)

    lo = _loss(fs, rs, ohf, ohr, Wd, bd.reshape(1, _NHID2), wc1p, bc1p,
               Wc2, bc2.reshape(1, _K))
    return (out1, lo[0, 0], lo[0, 1], h)


# speedup vs baseline: 9.4798x; 1.0249x over previous
import functools

import jax
import jax.numpy as jnp
import numpy as np
from jax import lax
from jax.experimental import pallas as pl
from jax.experimental.pallas import tpu as pltpu
from jax.experimental.pallas import tpu_sc as plsc

_N = 50000
_M = 16
_D = 64
_K = 8
_DD = 8
_NFEAT = 128
_NCLASS = 16
_NHID2 = 4
_NS = 160
_EPS = 1e-12

_B = 1000
_RB = 1000
_ZW = _M * _D

_NW = 32
_CHUNK = 1000
_NCHUNKS = (_N * _M) // _CHUNK
_CPW = _NCHUNKS // _NW


def _routing_mats():
    l = np.arange(_ZW)
    m = l // _D
    c = l % _D
    j_of_l = m * _K + c // _DD
    SS = np.zeros((_ZW, 128), np.float32)
    SS[l, j_of_l] = 1.0
    WE = SS.T.copy()
    T = np.zeros((_D, _ZW), np.float32)
    T[c, l] = 1.0
    R = T.T.copy()
    j = np.arange(128)
    A1 = (j[:, None] % _K == j[None, :] % _K).astype(np.float32)
    A2 = (j[:, None] // _K == j[None, :] // _K).astype(np.float32)
    return SS, WE, T, R, A1, A2


_SS, _WE, _T, _R, _A1, _A2 = _routing_mats()


def _seg_mats(width):
    g = width // _DD
    lane = lax.broadcasted_iota(jnp.int32, (width, g), 0)
    col = lax.broadcasted_iota(jnp.int32, (width, g), 1)
    S = (lane // _DD == col).astype(jnp.float32)
    row = lax.broadcasted_iota(jnp.int32, (g, width), 0)
    lane2 = lax.broadcasted_iota(jnp.int32, (g, width), 1)
    E = (lane2 // _DD == row).astype(jnp.float32)
    return S, E


def _gdot(a, b):
    return jnp.dot(a, b, preferred_element_type=jnp.float32,
                   precision=lax.Precision.HIGHEST)


def _ddot(a, b):
    return jnp.dot(a, b, preferred_element_type=jnp.float32)


def _gnorm(u):
    S, E = _seg_mats(u.shape[-1])
    s = _ddot(u * u, S)
    inv = 1.0 / jnp.maximum(jnp.sqrt(s), _EPS)
    return u * _ddot(inv, E)


def _pad128(v):
    return jnp.concatenate([v, jnp.zeros_like(v)], axis=-1)


def _pca_body(x_ref, w_ref, b_ref, h_ref, hn_ref):
    h = jnp.maximum(_ddot(x_ref[...], w_ref[...]) + b_ref[...], 0.0)
    h_ref[...] = h
    hn_ref[...] = _gnorm(h)


def _routing_body(param_ref, z_ref, u0_ref, ss_ref, we_ref, t_ref, r_ref,
                  a1_ref, a2_ref, h_ref, hn_ref):
    param = param_ref[0]
    z = z_ref[...]
    ub = u0_ref[...]
    SS = ss_ref[...]
    WE = we_ref[...]
    T = t_ref[...]
    R = r_ref[...]
    A1 = a1_ref[...]
    A2 = a2_ref[...]

    c0 = param / 16.0 + (1.0 - param) / 8.0
    u = c0 * _ddot(z, R) + ub
    u = _gnorm(u)

    for it in range(1, 3):
        ut = _ddot(u, T)
        p = _ddot(z * ut, SS)
        ep = jnp.exp(p)
        d1 = _ddot(ep, A1)
        d2 = _ddot(ep, A2)
        w = param * (ep / d1) + (1.0 - param) * (ep / d2)
        we = _ddot(w, WE)
        u = _ddot(z * we, R) + ub
        if it < 2:
            u = _gnorm(u)

    h = jnp.maximum(u, 0.0)
    h_ref[...] = h
    hn_ref[...] = _gnorm(h)


def _logit_body(h_ref, w_ref, b_ref, out_ref):
    logit = _ddot(h_ref[...], w_ref[...]) + b_ref[...]
    m = jnp.max(logit, axis=-1, keepdims=True)
    e = logit - m
    lse = jnp.log(jnp.sum(jnp.exp(e), axis=-1, keepdims=True))
    out_ref[...] = e - lse


def _loss_body(fs_ref, rs_ref, ohf_ref, ohr_ref, wd_ref, bd_ref,
               wc1_ref, bc1_ref, wc2_ref, bc2_ref, out_ref):
    hf = jnp.maximum(_gdot(fs_ref[...], wd_ref[...]) + bd_ref[...], 0.0)
    hr = jnp.maximum(_gdot(rs_ref[...], wd_ref[...]) + bd_ref[...], 0.0)
    d_fake = _gdot(hf, wc1_ref[...]) + bc1_ref[...]
    prob = _gdot(hr, wc2_ref[...]) + bc2_ref[...]

    t = -d_fake
    sp = jnp.maximum(t, 0.0) + jnp.log(1.0 + jnp.exp(-jnp.abs(t)))
    lane = lax.broadcasted_iota(jnp.int32, (1, _K), 1)
    g = jnp.sum(sp * (lane == 0).astype(jnp.float32)) / _NS

    m = jnp.max(prob, axis=-1, keepdims=True)
    e = prob - m
    ls = e - jnp.log(jnp.sum(jnp.exp(e), axis=-1, keepdims=True))
    cls_r = -jnp.sum(ls * ohr_ref[...]) / _NS
    cls_f = -jnp.sum(ls * ohf_ref[...]) / _NS

    out_ref[...] = (jnp.where(lane == 0, g + cls_r, 0.0)
                    + jnp.where(lane == 1, g + cls_f, 0.0))


_pca = pl.pallas_call(
    _pca_body,
    grid=(_N // _RB,),
    in_specs=[
        pl.BlockSpec((_RB, _NFEAT), lambda i: (i, 0)),
        pl.BlockSpec((_NFEAT, _D), lambda i: (0, 0)),
        pl.BlockSpec((1, _D), lambda i: (0, 0)),
    ],
    out_specs=[
        pl.BlockSpec((_RB, _D), lambda i: (i, 0)),
        pl.BlockSpec((_RB, _D), lambda i: (i, 0)),
    ],
    out_shape=[
        jax.ShapeDtypeStruct((_N, _D), jnp.float32),
        jax.ShapeDtypeStruct((_N, _D), jnp.float32),
    ],
    compiler_params=pltpu.CompilerParams(dimension_semantics=("parallel",)),
)


_routing = pl.pallas_call(
    _routing_body,
    grid=(_N // _B,),
    in_specs=[
        pl.BlockSpec(memory_space=pltpu.SMEM),
        pl.BlockSpec((_B, _ZW), lambda i: (i, 0)),
        pl.BlockSpec((_B, _D), lambda i: (i, 0)),
        pl.BlockSpec((_ZW, 128), lambda i: (0, 0)),
        pl.BlockSpec((128, _ZW), lambda i: (0, 0)),
        pl.BlockSpec((_D, _ZW), lambda i: (0, 0)),
        pl.BlockSpec((_ZW, _D), lambda i: (0, 0)),
        pl.BlockSpec((128, 128), lambda i: (0, 0)),
        pl.BlockSpec((128, 128), lambda i: (0, 0)),
    ],
    out_specs=[
        pl.BlockSpec((_B, _D), lambda i: (i, 0)),
        pl.BlockSpec((_B, _D), lambda i: (i, 0)),
    ],
    out_shape=[
        jax.ShapeDtypeStruct((_N, _D), jnp.float32),
        jax.ShapeDtypeStruct((_N, _D), jnp.float32),
    ],
    compiler_params=pltpu.CompilerParams(dimension_semantics=("parallel",)),
)


_logit = pl.pallas_call(
    _logit_body,
    grid=(_N // _RB,),
    in_specs=[
        pl.BlockSpec((_RB, _D), lambda i: (i, 0)),
        pl.BlockSpec((_D, _NCLASS), lambda i: (0, 0)),
        pl.BlockSpec((1, _NCLASS), lambda i: (0, 0)),
    ],
    out_specs=pl.BlockSpec((_RB, _NCLASS), lambda i: (i, 0)),
    out_shape=jax.ShapeDtypeStruct((_N, _NCLASS), jnp.float32),
    compiler_params=pltpu.CompilerParams(dimension_semantics=("parallel",)),
)


_loss = pl.pallas_call(
    _loss_body,
    out_shape=jax.ShapeDtypeStruct((1, _K), jnp.float32),
)


_sc_gather_cached = None


def _get_sc_gather():
    global _sc_gather_cached
    if _sc_gather_cached is not None:
        return _sc_gather_cached

    @functools.partial(
        pl.kernel,
        mesh=plsc.VectorSubcoreMesh(core_axis_name="c", subcore_axis_name="s"),
        out_type=jax.ShapeDtypeStruct((_N * _M, _D), jnp.float32),
        scratch_types=[
            pltpu.VMEM((_CHUNK,), jnp.int32),
            pltpu.VMEM((_CHUNK, _D), jnp.float32),
            pltpu.SemaphoreType.DMA,
        ],
        compiler_params=pltpu.CompilerParams(use_tc_tiling_on_sc=False),
    )
    def _sc_gather(tab_hbm, idx_hbm, out_hbm, idx_v, rows_v, sem):
        wid = lax.axis_index("s") * 2 + lax.axis_index("c")

        def body(i, carry):
            chunk = wid * _CPW + i
            base = chunk * _CHUNK
            pltpu.sync_copy(idx_hbm.at[pl.ds(base, _CHUNK)], idx_v)
            pltpu.async_copy(tab_hbm.at[idx_v], rows_v, sem).wait()
            pltpu.sync_copy(rows_v, out_hbm.at[pl.ds(base, _CHUNK)])
            return carry

        lax.fori_loop(0, _CPW, body, 0)

    _sc_gather_cached = _sc_gather
    return _sc_gather


def kernel(x, nb, W_pca, b_pca, param_p, W_mlp, b_mlp, Wd, bd, Wc1, bc1,
           Wc2, bc2, fake_node, fake_cap, real_node, real_cap):
    f32 = jnp.float32
    param = jax.nn.sigmoid(param_p.astype(f32))

    h0, hn = _pca(x, W_pca, b_pca.reshape(1, _D))
    idx = nb.astype(jnp.int32).reshape(-1)

    sc_gather = _get_sc_gather()
    h = h0
    for _ in range(2):
        z = sc_gather(hn, idx)
        h, hn = _routing(param, z.reshape(_N, _ZW), hn,
                         _SS, _WE, _T, _R, _A1, _A2)

    out1 = _logit(h, W_mlp, b_mlp.reshape(1, _NCLASS))

    fs = h0.reshape(_N, _K, _DD)[fake_node, fake_cap]
    rs = h.reshape(_N, _K, _DD)[real_node, real_cap]
    ohf = jax.nn.one_hot(fake_cap, _K, dtype=f32)
    ohr = jax.nn.one_hot(real_cap, _K, dtype=f32)
    wc1p = jnp.pad(Wc1, ((0, 0), (0, _K - 1)))
    bc1p = jnp.pad(bc1, (0, _K - 1)).reshape(1, _K)

    lo = _loss(fs, rs, ohf, ohr, Wd, bd.reshape(1, _NHID2), wc1p, bc1p,
               Wc2, bc2.reshape(1, _K))
    return (out1, lo[0, 0], lo[0, 1], h)


# speedup vs baseline: 9.8447x; 1.0385x over previous
import functools

import jax
import jax.numpy as jnp
import numpy as np
from jax import lax
from jax.experimental import pallas as pl
from jax.experimental.pallas import tpu as pltpu
from jax.experimental.pallas import tpu_sc as plsc

_N = 50000
_M = 16
_D = 64
_K = 8
_DD = 8
_NFEAT = 128
_NCLASS = 16
_NHID2 = 4
_NS = 160
_EPS = 1e-12

_B = 1000
_RB = 1000
_ZW = _M * _D
_NP = 5
_NH = _N // _NP

_NW = 32
_CHUNK = 1000
_NCHUNKS = (_NH * _M) // _CHUNK
_CPW = _NCHUNKS // _NW


def _routing_mats():
    l = np.arange(_ZW)
    m = l // _D
    c = l % _D
    j_of_l = m * _K + c // _DD
    SS = np.zeros((_ZW, 128), np.float32)
    SS[l, j_of_l] = 1.0
    WE = SS.T.copy()
    T = np.zeros((_D, _ZW), np.float32)
    T[c, l] = 1.0
    R = T.T.copy()
    j = np.arange(128)
    A1 = (j[:, None] % _K == j[None, :] % _K).astype(np.float32)
    A2 = (j[:, None] // _K == j[None, :] // _K).astype(np.float32)
    return SS, WE, T, R, A1, A2


_SS, _WE, _T, _R, _A1, _A2 = _routing_mats()


def _seg_mats(width):
    g = width // _DD
    lane = lax.broadcasted_iota(jnp.int32, (width, g), 0)
    col = lax.broadcasted_iota(jnp.int32, (width, g), 1)
    S = (lane // _DD == col).astype(jnp.float32)
    row = lax.broadcasted_iota(jnp.int32, (g, width), 0)
    lane2 = lax.broadcasted_iota(jnp.int32, (g, width), 1)
    E = (lane2 // _DD == row).astype(jnp.float32)
    return S, E


def _gdot(a, b):
    return jnp.dot(a, b, preferred_element_type=jnp.float32,
                   precision=lax.Precision.HIGHEST)


def _ddot(a, b):
    return jnp.dot(a, b, preferred_element_type=jnp.float32)


def _gnorm(u):
    S, E = _seg_mats(u.shape[-1])
    s = _ddot(u * u, S)
    inv = 1.0 / jnp.maximum(jnp.sqrt(s), _EPS)
    return u * _ddot(inv, E)


def _pad128(v):
    return jnp.concatenate([v, jnp.zeros_like(v)], axis=-1)


def _pca_body(x_ref, w_ref, b_ref, h_ref, hn_ref):
    h = jnp.maximum(_ddot(x_ref[...], w_ref[...]) + b_ref[...], 0.0)
    h_ref[...] = h
    hn_ref[...] = _gnorm(h)


def _routing_body(param_ref, z_ref, u0_ref, ss_ref, we_ref, t_ref, r_ref,
                  a1_ref, a2_ref, h_ref, hn_ref):
    param = param_ref[0]
    z = z_ref[...]
    ub = u0_ref[...]
    SS = ss_ref[...]
    WE = we_ref[...]
    T = t_ref[...]
    R = r_ref[...]
    A1 = a1_ref[...]
    A2 = a2_ref[...]

    c0 = param / 16.0 + (1.0 - param) / 8.0
    u = c0 * _ddot(z, R) + ub
    u = _gnorm(u)

    for it in range(1, 3):
        ut = _ddot(u, T)
        p = _ddot(z * ut, SS)
        ep = jnp.exp(p)
        d1 = _ddot(ep, A1)
        d2 = _ddot(ep, A2)
        w = param * (ep / d1) + (1.0 - param) * (ep / d2)
        we = _ddot(w, WE)
        u = _ddot(z * we, R) + ub
        if it < 2:
            u = _gnorm(u)

    h = jnp.maximum(u, 0.0)
    h_ref[...] = h
    hn_ref[...] = _gnorm(h)


def _logit_body(h_ref, w_ref, b_ref, out_ref):
    logit = _ddot(h_ref[...], w_ref[...]) + b_ref[...]
    m = jnp.max(logit, axis=-1, keepdims=True)
    e = logit - m
    lse = jnp.log(jnp.sum(jnp.exp(e), axis=-1, keepdims=True))
    out_ref[...] = e - lse


def _loss_body(fs_ref, rs_ref, ohf_ref, ohr_ref, wd_ref, bd_ref,
               wc1_ref, bc1_ref, wc2_ref, bc2_ref, out_ref):
    hf = jnp.maximum(_gdot(fs_ref[...], wd_ref[...]) + bd_ref[...], 0.0)
    hr = jnp.maximum(_gdot(rs_ref[...], wd_ref[...]) + bd_ref[...], 0.0)
    d_fake = _gdot(hf, wc1_ref[...]) + bc1_ref[...]
    prob = _gdot(hr, wc2_ref[...]) + bc2_ref[...]

    t = -d_fake
    sp = jnp.maximum(t, 0.0) + jnp.log(1.0 + jnp.exp(-jnp.abs(t)))
    lane = lax.broadcasted_iota(jnp.int32, (1, _K), 1)
    g = jnp.sum(sp * (lane == 0).astype(jnp.float32)) / _NS

    m = jnp.max(prob, axis=-1, keepdims=True)
    e = prob - m
    ls = e - jnp.log(jnp.sum(jnp.exp(e), axis=-1, keepdims=True))
    cls_r = -jnp.sum(ls * ohr_ref[...]) / _NS
    cls_f = -jnp.sum(ls * ohf_ref[...]) / _NS

    out_ref[...] = (jnp.where(lane == 0, g + cls_r, 0.0)
                    + jnp.where(lane == 1, g + cls_f, 0.0))


_pca = pl.pallas_call(
    _pca_body,
    grid=(_N // _RB,),
    in_specs=[
        pl.BlockSpec((_RB, _NFEAT), lambda i: (i, 0)),
        pl.BlockSpec((_NFEAT, _D), lambda i: (0, 0)),
        pl.BlockSpec((1, _D), lambda i: (0, 0)),
    ],
    out_specs=[
        pl.BlockSpec((_RB, _D), lambda i: (i, 0)),
        pl.BlockSpec((_RB, _D), lambda i: (i, 0)),
    ],
    out_shape=[
        jax.ShapeDtypeStruct((_N, _D), jnp.float32),
        jax.ShapeDtypeStruct((_N, _D), jnp.float32),
    ],
    compiler_params=pltpu.CompilerParams(dimension_semantics=("parallel",)),
)


_routing = pl.pallas_call(
    _routing_body,
    grid=(_NH // _B,),
    in_specs=[
        pl.BlockSpec(memory_space=pltpu.SMEM),
        pl.BlockSpec((_B, _ZW), lambda i: (i, 0)),
        pl.BlockSpec((_B, _D), lambda i: (i, 0)),
        pl.BlockSpec((_ZW, 128), lambda i: (0, 0)),
        pl.BlockSpec((128, _ZW), lambda i: (0, 0)),
        pl.BlockSpec((_D, _ZW), lambda i: (0, 0)),
        pl.BlockSpec((_ZW, _D), lambda i: (0, 0)),
        pl.BlockSpec((128, 128), lambda i: (0, 0)),
        pl.BlockSpec((128, 128), lambda i: (0, 0)),
    ],
    out_specs=[
        pl.BlockSpec((_B, _D), lambda i: (i, 0)),
        pl.BlockSpec((_B, _D), lambda i: (i, 0)),
    ],
    out_shape=[
        jax.ShapeDtypeStruct((_NH, _D), jnp.float32),
        jax.ShapeDtypeStruct((_NH, _D), jnp.float32),
    ],
    compiler_params=pltpu.CompilerParams(dimension_semantics=("parallel",)),
)


_logit = pl.pallas_call(
    _logit_body,
    grid=(_N // _RB,),
    in_specs=[
        pl.BlockSpec((_RB, _D), lambda i: (i, 0)),
        pl.BlockSpec((_D, _NCLASS), lambda i: (0, 0)),
        pl.BlockSpec((1, _NCLASS), lambda i: (0, 0)),
    ],
    out_specs=pl.BlockSpec((_RB, _NCLASS), lambda i: (i, 0)),
    out_shape=jax.ShapeDtypeStruct((_N, _NCLASS), jnp.float32),
    compiler_params=pltpu.CompilerParams(dimension_semantics=("parallel",)),
)


_loss = pl.pallas_call(
    _loss_body,
    out_shape=jax.ShapeDtypeStruct((1, _K), jnp.float32),
)


_sc_gather_cached = None


def _get_sc_gather():
    global _sc_gather_cached
    if _sc_gather_cached is not None:
        return _sc_gather_cached

    @functools.partial(
        pl.kernel,
        mesh=plsc.VectorSubcoreMesh(core_axis_name="c", subcore_axis_name="s"),
        out_type=jax.ShapeDtypeStruct((_NH * _M, _D), jnp.float32),
        scratch_types=[
            pltpu.VMEM((_CHUNK,), jnp.int32),
            pltpu.VMEM((_CHUNK, _D), jnp.float32),
            pltpu.SemaphoreType.DMA,
        ],
        compiler_params=pltpu.CompilerParams(use_tc_tiling_on_sc=False),
    )
    def _sc_gather(tab_hbm, idx_hbm, out_hbm, idx_v, rows_v, sem):
        wid = lax.axis_index("s") * 2 + lax.axis_index("c")

        def body(i, carry):
            chunk = wid * _CPW + i
            base = chunk * _CHUNK
            pltpu.sync_copy(idx_hbm.at[pl.ds(base, _CHUNK)], idx_v)
            pltpu.async_copy(tab_hbm.at[idx_v], rows_v, sem).wait()
            pltpu.sync_copy(rows_v, out_hbm.at[pl.ds(base, _CHUNK)])
            return carry

        lax.fori_loop(0, _CPW, body, 0)

    _sc_gather_cached = _sc_gather
    return _sc_gather


def kernel(x, nb, W_pca, b_pca, param_p, W_mlp, b_mlp, Wd, bd, Wc1, bc1,
           Wc2, bc2, fake_node, fake_cap, real_node, real_cap):
    f32 = jnp.float32
    param = jax.nn.sigmoid(param_p.astype(f32))

    h0, hn = _pca(x, W_pca, b_pca.reshape(1, _D))
    idx = nb.astype(jnp.int32).reshape(-1)

    sc_gather = _get_sc_gather()
    idx_parts = [idx[p * _NH * _M:(p + 1) * _NH * _M] for p in range(_NP)]
    h = h0
    for _ in range(2):
        zs = [sc_gather(hn, idx_parts[p]) for p in range(_NP)]
        outs = [_routing(param, zs[p].reshape(_NH, _ZW),
                         lax.slice_in_dim(hn, p * _NH, (p + 1) * _NH),
                         _SS, _WE, _T, _R, _A1, _A2) for p in range(_NP)]
        h = jnp.concatenate([o[0] for o in outs], axis=0)
        hn = jnp.concatenate([o[1] for o in outs], axis=0)

    out1 = _logit(h, W_mlp, b_mlp.reshape(1, _NCLASS))

    fs = h0.reshape(_N, _K, _DD)[fake_node, fake_cap]
    rs = h.reshape(_N, _K, _DD)[real_node, real_cap]
    ohf = jax.nn.one_hot(fake_cap, _K, dtype=f32)
    ohr = jax.nn.one_hot(real_cap, _K, dtype=f32)
    wc1p = jnp.pad(Wc1, ((0, 0), (0, _K - 1)))
    bc1p = jnp.pad(bc1, (0, _K - 1)).reshape(1, _K)

    lo = _loss(fs, rs, ohf, ohr, Wd, bd.reshape(1, _NHID2), wc1p, bc1p,
               Wc2, bc2.reshape(1, _K))
    return (out1, lo[0, 0], lo[0, 1], h)


# speedup vs baseline: 9.9514x; 1.0108x over previous
import functools

import jax
import jax.numpy as jnp
import numpy as np
from jax import lax
from jax.experimental import pallas as pl
from jax.experimental.pallas import tpu as pltpu
from jax.experimental.pallas import tpu_sc as plsc

_N = 50000
_M = 16
_D = 64
_K = 8
_DD = 8
_NFEAT = 128
_NCLASS = 16
_NHID2 = 4
_NS = 160
_EPS = 1e-12

_B = 1000
_RB = 1000
_ZW = _M * _D
_NP = 5
_NH = _N // _NP

_NW = 32
_CHUNK = 1000
_NCHUNKS = (_NH * _M) // _CHUNK
_CPW = _NCHUNKS // _NW


def _routing_mats():
    l = np.arange(_ZW)
    m = l // _D
    c = l % _D
    j_of_l = m * _K + c // _DD
    SS = np.zeros((_ZW, 128), np.float32)
    SS[l, j_of_l] = 1.0
    WE = SS.T.copy()
    T = np.zeros((_D, _ZW), np.float32)
    T[c, l] = 1.0
    R = T.T.copy()
    j = np.arange(128)
    A1 = (j[:, None] % _K == j[None, :] % _K).astype(np.float32)
    A2 = (j[:, None] // _K == j[None, :] // _K).astype(np.float32)
    return SS, WE, T, R, A1, A2


_SS, _WE, _T, _R, _A1, _A2 = _routing_mats()


def _seg_mats(width):
    g = width // _DD
    lane = lax.broadcasted_iota(jnp.int32, (width, g), 0)
    col = lax.broadcasted_iota(jnp.int32, (width, g), 1)
    S = (lane // _DD == col).astype(jnp.float32)
    row = lax.broadcasted_iota(jnp.int32, (g, width), 0)
    lane2 = lax.broadcasted_iota(jnp.int32, (g, width), 1)
    E = (lane2 // _DD == row).astype(jnp.float32)
    return S, E


def _gdot(a, b):
    return jnp.dot(a, b, preferred_element_type=jnp.float32,
                   precision=lax.Precision.HIGHEST)


def _ddot(a, b):
    return jnp.dot(a, b, preferred_element_type=jnp.float32)


def _bdot(a, b):
    return jnp.dot(a.astype(jnp.bfloat16), b.astype(jnp.bfloat16),
                   preferred_element_type=jnp.float32)


def _gnorm(u):
    S, E = _seg_mats(u.shape[-1])
    s = _ddot(u * u, S)
    inv = 1.0 / jnp.maximum(jnp.sqrt(s), _EPS)
    return u * _ddot(inv, E)


def _pad128(v):
    return jnp.concatenate([v, jnp.zeros_like(v)], axis=-1)


def _pca_body(x_ref, w_ref, b_ref, h_ref, hn_ref):
    h = jnp.maximum(_ddot(x_ref[...], w_ref[...]) + b_ref[...], 0.0)
    h_ref[...] = h
    hn_ref[...] = _gnorm(h)


def _routing_body(param_ref, z_ref, u0_ref, ss_ref, we_ref, t_ref, r_ref,
                  a1_ref, a2_ref, h_ref, hn_ref):
    param = param_ref[0]
    z = z_ref[...]
    ub = u0_ref[...]
    SS = ss_ref[...]
    WE = we_ref[...]
    T = t_ref[...]
    R = r_ref[...]
    A1 = a1_ref[...]
    A2 = a2_ref[...]

    c0 = param / 16.0 + (1.0 - param) / 8.0
    u = c0 * _ddot(z, R) + ub
    u = _gnorm(u)

    for it in range(1, 3):
        ut = jnp.tile(u, (1, _M))
        p = _bdot(z * ut, SS)
        ep = jnp.exp(p)
        d1 = _bdot(ep, A1)
        d2 = _bdot(ep, A2)
        w = param * (ep / d1) + (1.0 - param) * (ep / d2)
        we = _ddot(w, WE)
        u = _ddot(z * we, R) + ub
        if it < 2:
            u = _gnorm(u)

    h = jnp.maximum(u, 0.0)
    h_ref[...] = h
    hn_ref[...] = _gnorm(h)


def _logit_body(h_ref, w_ref, b_ref, out_ref):
    logit = _ddot(h_ref[...], w_ref[...]) + b_ref[...]
    m = jnp.max(logit, axis=-1, keepdims=True)
    e = logit - m
    lse = jnp.log(jnp.sum(jnp.exp(e), axis=-1, keepdims=True))
    out_ref[...] = e - lse


def _loss_body(fs_ref, rs_ref, ohf_ref, ohr_ref, wd_ref, bd_ref,
               wc1_ref, bc1_ref, wc2_ref, bc2_ref, out_ref):
    hf = jnp.maximum(_gdot(fs_ref[...], wd_ref[...]) + bd_ref[...], 0.0)
    hr = jnp.maximum(_gdot(rs_ref[...], wd_ref[...]) + bd_ref[...], 0.0)
    d_fake = _gdot(hf, wc1_ref[...]) + bc1_ref[...]
    prob = _gdot(hr, wc2_ref[...]) + bc2_ref[...]

    t = -d_fake
    sp = jnp.maximum(t, 0.0) + jnp.log(1.0 + jnp.exp(-jnp.abs(t)))
    lane = lax.broadcasted_iota(jnp.int32, (1, _K), 1)
    g = jnp.sum(sp * (lane == 0).astype(jnp.float32)) / _NS

    m = jnp.max(prob, axis=-1, keepdims=True)
    e = prob - m
    ls = e - jnp.log(jnp.sum(jnp.exp(e), axis=-1, keepdims=True))
    cls_r = -jnp.sum(ls * ohr_ref[...]) / _NS
    cls_f = -jnp.sum(ls * ohf_ref[...]) / _NS

    out_ref[...] = (jnp.where(lane == 0, g + cls_r, 0.0)
                    + jnp.where(lane == 1, g + cls_f, 0.0))


_pca = pl.pallas_call(
    _pca_body,
    grid=(_N // _RB,),
    in_specs=[
        pl.BlockSpec((_RB, _NFEAT), lambda i: (i, 0)),
        pl.BlockSpec((_NFEAT, _D), lambda i: (0, 0)),
        pl.BlockSpec((1, _D), lambda i: (0, 0)),
    ],
    out_specs=[
        pl.BlockSpec((_RB, _D), lambda i: (i, 0)),
        pl.BlockSpec((_RB, _D), lambda i: (i, 0)),
    ],
    out_shape=[
        jax.ShapeDtypeStruct((_N, _D), jnp.float32),
        jax.ShapeDtypeStruct((_N, _D), jnp.float32),
    ],
    compiler_params=pltpu.CompilerParams(dimension_semantics=("parallel",)),
)


_routing = pl.pallas_call(
    _routing_body,
    grid=(_NH // _B,),
    in_specs=[
        pl.BlockSpec(memory_space=pltpu.SMEM),
        pl.BlockSpec((_B, _ZW), lambda i: (i, 0)),
        pl.BlockSpec((_B, _D), lambda i: (i, 0)),
        pl.BlockSpec((_ZW, 128), lambda i: (0, 0)),
        pl.BlockSpec((128, _ZW), lambda i: (0, 0)),
        pl.BlockSpec((_D, _ZW), lambda i: (0, 0)),
        pl.BlockSpec((_ZW, _D), lambda i: (0, 0)),
        pl.BlockSpec((128, 128), lambda i: (0, 0)),
        pl.BlockSpec((128, 128), lambda i: (0, 0)),
    ],
    out_specs=[
        pl.BlockSpec((_B, _D), lambda i: (i, 0)),
        pl.BlockSpec((_B, _D), lambda i: (i, 0)),
    ],
    out_shape=[
        jax.ShapeDtypeStruct((_NH, _D), jnp.float32),
        jax.ShapeDtypeStruct((_NH, _D), jnp.float32),
    ],
    compiler_params=pltpu.CompilerParams(dimension_semantics=("parallel",)),
)


_logit = pl.pallas_call(
    _logit_body,
    grid=(_N // _RB,),
    in_specs=[
        pl.BlockSpec((_RB, _D), lambda i: (i, 0)),
        pl.BlockSpec((_D, _NCLASS), lambda i: (0, 0)),
        pl.BlockSpec((1, _NCLASS), lambda i: (0, 0)),
    ],
    out_specs=pl.BlockSpec((_RB, _NCLASS), lambda i: (i, 0)),
    out_shape=jax.ShapeDtypeStruct((_N, _NCLASS), jnp.float32),
    compiler_params=pltpu.CompilerParams(dimension_semantics=("parallel",)),
)


_loss = pl.pallas_call(
    _loss_body,
    out_shape=jax.ShapeDtypeStruct((1, _K), jnp.float32),
)


_sc_gather_cached = None


def _get_sc_gather():
    global _sc_gather_cached
    if _sc_gather_cached is not None:
        return _sc_gather_cached

    @functools.partial(
        pl.kernel,
        mesh=plsc.VectorSubcoreMesh(core_axis_name="c", subcore_axis_name="s"),
        out_type=jax.ShapeDtypeStruct((_NH * _M, _D), jnp.float32),
        scratch_types=[
            pltpu.VMEM((_CHUNK,), jnp.int32),
            pltpu.VMEM((_CHUNK, _D), jnp.float32),
            pltpu.SemaphoreType.DMA,
        ],
        compiler_params=pltpu.CompilerParams(use_tc_tiling_on_sc=False),
    )
    def _sc_gather(tab_hbm, idx_hbm, out_hbm, idx_v, rows_v, sem):
        wid = lax.axis_index("s") * 2 + lax.axis_index("c")

        def body(i, carry):
            chunk = wid * _CPW + i
            base = chunk * _CHUNK
            pltpu.sync_copy(idx_hbm.at[pl.ds(base, _CHUNK)], idx_v)
            pltpu.async_copy(tab_hbm.at[idx_v], rows_v, sem).wait()
            pltpu.sync_copy(rows_v, out_hbm.at[pl.ds(base, _CHUNK)])
            return carry

        lax.fori_loop(0, _CPW, body, 0)

    _sc_gather_cached = _sc_gather
    return _sc_gather


def kernel(x, nb, W_pca, b_pca, param_p, W_mlp, b_mlp, Wd, bd, Wc1, bc1,
           Wc2, bc2, fake_node, fake_cap, real_node, real_cap):
    f32 = jnp.float32
    param = jax.nn.sigmoid(param_p.astype(f32))

    h0, hn = _pca(x, W_pca, b_pca.reshape(1, _D))
    idx = nb.astype(jnp.int32).reshape(-1)

    sc_gather = _get_sc_gather()
    idx_parts = [idx[p * _NH * _M:(p + 1) * _NH * _M] for p in range(_NP)]
    h = h0
    for _ in range(2):
        zs = [sc_gather(hn, idx_parts[p]) for p in range(_NP)]
        outs = [_routing(param, zs[p].reshape(_NH, _ZW),
                         lax.slice_in_dim(hn, p * _NH, (p + 1) * _NH),
                         _SS, _WE, _T, _R, _A1, _A2) for p in range(_NP)]
        h = jnp.concatenate([o[0] for o in outs], axis=0)
        hn = jnp.concatenate([o[1] for o in outs], axis=0)

    out1 = _logit(h, W_mlp, b_mlp.reshape(1, _NCLASS))

    fs = h0.reshape(_N, _K, _DD)[fake_node, fake_cap]
    rs = h.reshape(_N, _K, _DD)[real_node, real_cap]
    ohf = jax.nn.one_hot(fake_cap, _K, dtype=f32)
    ohr = jax.nn.one_hot(real_cap, _K, dtype=f32)
    wc1p = jnp.pad(Wc1, ((0, 0), (0, _K - 1)))
    bc1p = jnp.pad(bc1, (0, _K - 1)).reshape(1, _K)

    lo = _loss(fs, rs, ohf, ohr, Wd, bd.reshape(1, _NHID2), wc1p, bc1p,
               Wc2, bc2.reshape(1, _K))
    return (out1, lo[0, 0], lo[0, 1], h)
